# baseline (speedup 1.0000x reference)
import functools

import jax
import jax.numpy as jnp
from jax import lax
from jax.experimental import pallas as pl
from jax.experimental.pallas import tpu as pltpu
from jax.experimental.pallas import tpu_sc as plsc

S = 2048
D = 1024
H = 16
DH = 64
E = 8
K = 2
F = 512
EPS = 1e-6

TQ = 256
TM = 256
NPAD = 6144
NT = NPAD // TM

_INTERPRET = False


def _rms(x, g):
    v = jnp.mean(x * x, axis=-1, keepdims=True)
    return x * lax.rsqrt(v + EPS) * g


def _qkv_kernel(h_ref, g_ref, w_ref, qkv_ref):
    x = _rms(h_ref[...], g_ref[...])
    qkv_ref[...] = jnp.dot(x.astype(jnp.bfloat16), w_ref[...],
                           preferred_element_type=jnp.float32).astype(jnp.bfloat16)


def _qkv(hidden, gamma, Wqkv_b):
    return pl.pallas_call(
        _qkv_kernel,
        grid=(S // TQ,),
        in_specs=[
            pl.BlockSpec((TQ, D), lambda i: (i, 0)),
            pl.BlockSpec((1, D), lambda i: (0, 0)),
            pl.BlockSpec((D, 3 * D), lambda i: (0, 0)),
        ],
        out_specs=pl.BlockSpec((TQ, 3 * D), lambda i: (i, 0)),
        out_shape=jax.ShapeDtypeStruct((S, 3 * D), jnp.bfloat16),
        interpret=_INTERPRET,
    )(hidden, gamma, Wqkv_b)


KC = 256


def _attn_kernel(q_ref, kv_ref, o_ref, m_ref, l_ref, acc_ref):
    qb = pl.program_id(0)
    kb = pl.program_id(1)

    @pl.when(kb == 0)
    def _init():
        m_ref[...] = jnp.full_like(m_ref, jnp.finfo(jnp.float32).min)
        l_ref[...] = jnp.zeros_like(l_ref)
        acc_ref[...] = jnp.zeros_like(acc_ref)

    @pl.when(kb <= qb)
    def _compute():
        rows = qb * TQ + lax.broadcasted_iota(jnp.int32, (TQ, KC), 0)
        cols = kb * KC + lax.broadcasted_iota(jnp.int32, (TQ, KC), 1)
        mask = cols <= rows
        for h in range(H):
            q = q_ref[:, h * DH:(h + 1) * DH]
            k = kv_ref[:, D + h * DH:D + (h + 1) * DH]
            v = kv_ref[:, 2 * D + h * DH:2 * D + (h + 1) * DH]
            s = lax.dot_general(q, k, (((1,), (1,)), ((), ())),
                                preferred_element_type=jnp.float32) * (1.0 / 8.0)
            s = jnp.where(mask, s, jnp.finfo(jnp.float32).min)
            m = m_ref[:, h:h + 1]
            m2 = jnp.maximum(m, jnp.max(s, axis=-1, keepdims=True))
            p = jnp.exp(s - m2)
            scale = jnp.exp(m - m2)
            m_ref[:, h:h + 1] = m2
            l_ref[:, h:h + 1] = (l_ref[:, h:h + 1] * scale
                                 + jnp.sum(p, axis=-1, keepdims=True))
            acc_ref[:, h * DH:(h + 1) * DH] = (
                acc_ref[:, h * DH:(h + 1) * DH] * scale
                + jnp.dot(p.astype(jnp.bfloat16), v,
                          preferred_element_type=jnp.float32))

    @pl.when(kb == qb)
    def _emit():
        for h in range(H):
            o_ref[:, h * DH:(h + 1) * DH] = (
                acc_ref[:, h * DH:(h + 1) * DH]
                / l_ref[:, h:h + 1]).astype(jnp.bfloat16)


def _attn(qkv):
    return pl.pallas_call(
        _attn_kernel,
        grid=(S // TQ, S // KC),
        in_specs=[
            pl.BlockSpec((TQ, D), lambda qb, kb: (qb, 0)),
            pl.BlockSpec((KC, 3 * D), lambda qb, kb: (jnp.minimum(kb, qb), 0)),
        ],
        out_specs=pl.BlockSpec((TQ, D), lambda qb, kb: (qb, 0)),
        out_shape=jax.ShapeDtypeStruct((S, D), jnp.bfloat16),
        scratch_shapes=[pltpu.VMEM((TQ, H), jnp.float32),
                        pltpu.VMEM((TQ, H), jnp.float32),
                        pltpu.VMEM((TQ, D), jnp.float32)],
        interpret=_INTERPRET,
    )(qkv, qkv)


def _oproj_kernel(a_ref, res_ref, wo_ref, g_ref, wg_ref,
                  h2_ref, hn_ref, sel_ref, w_ref, rank_ref, cnt_ref, run_ref):
    step = pl.program_id(0)

    @pl.when(step == 0)
    def _init():
        run_ref[...] = jnp.zeros_like(run_ref)

    o = jnp.dot(a_ref[...].astype(jnp.bfloat16), wo_ref[...],
                preferred_element_type=jnp.float32)
    h2 = res_ref[...] + o
    h2_ref[...] = h2
    hn = _rms(h2, g_ref[...])
    hn_ref[...] = hn
    logits = jnp.dot(hn.astype(jnp.bfloat16), wg_ref[...].astype(jnp.bfloat16),
                     preferred_element_type=jnp.float32)
    m = jnp.max(logits, axis=-1, keepdims=True)
    p = jnp.exp(logits - m)
    probs = p / jnp.sum(p, axis=-1, keepdims=True)
    idx = lax.broadcasted_iota(jnp.int32, (TQ, E), 1)
    m1 = jnp.max(probs, axis=-1, keepdims=True)
    i1 = jnp.min(jnp.where(probs == m1, idx, E), axis=-1, keepdims=True)
    probs2 = jnp.where(idx == i1, -1.0, probs)
    m2 = jnp.max(probs2, axis=-1, keepdims=True)
    i2 = jnp.min(jnp.where(probs2 == m2, idx, E), axis=-1, keepdims=True)
    tot = m1 + m2
    w1 = m1 / tot
    w2 = m2 / tot
    sel_ref[...] = jnp.where(idx == 0, i1, jnp.where(idx == 1, i2, 0))
    w_ref[...] = jnp.where(idx == 0, w1, jnp.where(idx == 1, w2, 0.0))

    oh1 = (idx == i1).astype(jnp.float32)
    oh2 = (idx == i2).astype(jnp.float32)
    oh = (oh1 + oh2).astype(jnp.bfloat16)
    r = lax.broadcasted_iota(jnp.int32, (TQ, TQ), 0)
    c = lax.broadcasted_iota(jnp.int32, (TQ, TQ), 1)
    tri = (c < r).astype(jnp.bfloat16)
    excl = jnp.dot(tri, oh, preferred_element_type=jnp.float32)
    run = run_ref[...]
    rank1 = jnp.sum((excl + run) * oh1, axis=-1, keepdims=True)
    rank2 = jnp.sum((excl + run) * oh2, axis=-1, keepdims=True)
    rank_ref[...] = jnp.where(idx == 0, rank1, jnp.where(idx == 1, rank2, 0.0))
    new_run = run + jnp.sum(oh1 + oh2, axis=0, keepdims=True)
    run_ref[...] = new_run
    cnt_ref[...] = new_run


def _oproj_router(attn, hidden, Wo_b, gamma, Wg):
    return pl.pallas_call(
        _oproj_kernel,
        grid=(S // TQ,),
        in_specs=[
            pl.BlockSpec((TQ, D), lambda i: (i, 0)),
            pl.BlockSpec((TQ, D), lambda i: (i, 0)),
            pl.BlockSpec((D, D), lambda i: (0, 0)),
            pl.BlockSpec((1, D), lambda i: (0, 0)),
            pl.BlockSpec((D, E), lambda i: (0, 0)),
        ],
        out_specs=[
            pl.BlockSpec((TQ, D), lambda i: (i, 0)),
            pl.BlockSpec((TQ, D), lambda i: (i, 0)),
            pl.BlockSpec((TQ, E), lambda i: (i, 0)),
            pl.BlockSpec((TQ, E), lambda i: (i, 0)),
            pl.BlockSpec((TQ, E), lambda i: (i, 0)),
            pl.BlockSpec((1, E), lambda i: (0, 0)),
        ],
        out_shape=[
            jax.ShapeDtypeStruct((S, D), jnp.float32),
            jax.ShapeDtypeStruct((S, D), jnp.float32),
            jax.ShapeDtypeStruct((S, E), jnp.int32),
            jax.ShapeDtypeStruct((S, E), jnp.float32),
            jax.ShapeDtypeStruct((S, E), jnp.float32),
            jax.ShapeDtypeStruct((1, E), jnp.float32),
        ],
        scratch_shapes=[pltpu.VMEM((1, E), jnp.float32)],
        interpret=_INTERPRET,
    )(attn, hidden, Wo_b, gamma, Wg)


def _gmm_kernel(te_ref, fill_ref, xs_ref, w1_ref, w2_ref, ys_ref):
    del te_ref
    t = pl.program_id(0)
    x = xs_ref[...].astype(jnp.bfloat16)
    gu = jnp.dot(x, w1_ref[...], preferred_element_type=jnp.float32)
    g = gu[:, :F]
    u = gu[:, F:]
    act = (g / (1.0 + jnp.exp(-g))) * u
    y = jnp.dot(act.astype(jnp.bfloat16), w2_ref[...],
                preferred_element_type=jnp.float32)
    valid = lax.broadcasted_iota(jnp.int32, (TM, 1), 0) < fill_ref[t]
    ys_ref[...] = jnp.where(valid, y, 0.0)


def _gmm(tile_expert, tile_fill, xs, W1_b, W2_b):
    grid_spec = pltpu.PrefetchScalarGridSpec(
        num_scalar_prefetch=2,
        grid=(NT,),
        in_specs=[
            pl.BlockSpec((TM, D), lambda t, te, fl: (t, 0)),
            pl.BlockSpec((D, 2 * F), lambda t, te, fl: (te[t], 0)),
            pl.BlockSpec((F, D), lambda t, te, fl: (te[t], 0)),
        ],
        out_specs=pl.BlockSpec((TM, D), lambda t, te, fl: (t, 0)),
    )
    return pl.pallas_call(
        _gmm_kernel,
        grid_spec=grid_spec,
        out_shape=jax.ShapeDtypeStruct((NPAD, D), jnp.float32),
        interpret=_INTERPRET,
    )(tile_expert, tile_fill, xs,
      W1_b.reshape(E * D, 2 * F), W2_b.reshape(E * F, D))


def _routing_plan(sel, rank, counts):
    counts_i = counts.reshape(E).astype(jnp.int32)
    gpad = ((counts_i + TM - 1) // TM) * TM
    startp = jnp.concatenate([jnp.zeros((1,), jnp.int32),
                              jnp.cumsum(gpad)[:-1].astype(jnp.int32)])
    pos0 = startp[sel[:, 0]] + rank[:, 0].astype(jnp.int32)
    pos1 = startp[sel[:, 1]] + rank[:, 1].astype(jnp.int32)
    tstart = jnp.arange(NT, dtype=jnp.int32) * TM
    te = jnp.sum(tstart[:, None] >= startp[None, :], axis=1) - 1
    te = jnp.clip(te, 0, E - 1).astype(jnp.int32)
    fill = jnp.clip(counts_i[te] - (tstart - startp[te]), 0, TM).astype(jnp.int32)
    return pos0, pos1, te, fill


NC = 2
NS = 16
NW = NC * NS
CHUNK = S // NW
CB = 32


def _dispatch_sc(hn, pos0, pos1):
    mesh = plsc.VectorSubcoreMesh(core_axis_name="c", subcore_axis_name="s")

    @functools.partial(
        pl.kernel, mesh=mesh,
        out_type=jax.ShapeDtypeStruct((NPAD, D), jnp.float32),
        scratch_types=[pltpu.VMEM((CHUNK,), jnp.int32),
                       pltpu.VMEM((CHUNK,), jnp.int32),
                       pltpu.VMEM((CHUNK, D), jnp.float32),
                       pltpu.SemaphoreType.DMA],
    )
    def k(hn_hbm, p0_hbm, p1_hbm, xs_hbm, i0_v, i1_v, rows_v, sem):
        wid = lax.axis_index("s") * NC + lax.axis_index("c")
        base = wid * CHUNK
        pltpu.sync_copy(p0_hbm.at[pl.ds(base, CHUNK)], i0_v)
        pltpu.sync_copy(p1_hbm.at[pl.ds(base, CHUNK)], i1_v)
        pltpu.sync_copy(hn_hbm.at[pl.ds(base, CHUNK)], rows_v)
        c0 = pltpu.async_copy(rows_v, xs_hbm.at[i0_v], sem)
        c1 = pltpu.async_copy(rows_v, xs_hbm.at[i1_v], sem)
        c0.wait()
        c1.wait()

    return k(hn, pos0, pos1)


def _combine_sc(h2, ys, pos0, pos1, w0, w1):
    mesh = plsc.VectorSubcoreMesh(core_axis_name="c", subcore_axis_name="s")

    @functools.partial(
        pl.kernel, mesh=mesh,
        out_type=jax.ShapeDtypeStruct((S, D), jnp.float32),
        scratch_types=[pltpu.VMEM((CB,), jnp.int32),
                       pltpu.VMEM((CB,), jnp.int32),
                       pltpu.VMEM((CB, 16), jnp.float32),
                       pltpu.VMEM((CB, 16), jnp.float32),
                       pltpu.VMEM((CB, D), jnp.float32),
                       pltpu.VMEM((CB, D), jnp.float32),
                       pltpu.VMEM((CB, D), jnp.float32),
                       pltpu.SemaphoreType.DMA],
    )
    def k(h2_hbm, ys_hbm, p0_hbm, p1_hbm, w0_hbm, w1_hbm, out_hbm,
          i0_v, i1_v, w0_v, w1_v, a_v, b_v, c_v, sem):
        wid = lax.axis_index("s") * NC + lax.axis_index("c")
        for sub in range(CHUNK // CB):
            base = wid * CHUNK + sub * CB
            pltpu.sync_copy(p0_hbm.at[pl.ds(base, CB)], i0_v)
            pltpu.sync_copy(p1_hbm.at[pl.ds(base, CB)], i1_v)
            pltpu.sync_copy(w0_hbm.at[pl.ds(base, CB)], w0_v)
            pltpu.sync_copy(w1_hbm.at[pl.ds(base, CB)], w1_v)
            g0 = pltpu.async_copy(ys_hbm.at[i0_v], a_v, sem)
            g1 = pltpu.async_copy(ys_hbm.at[i1_v], b_v, sem)
            pltpu.sync_copy(h2_hbm.at[pl.ds(base, CB)], c_v)
            g0.wait()
            g1.wait()

            def body(i, carry):
                wa = w0_v[i, :]
                wb = w1_v[i, :]
                for j in range(0, D, 16):
                    c_v[i, pl.ds(j, 16)] = (c_v[i, pl.ds(j, 16)]
                                            + wa * a_v[i, pl.ds(j, 16)]
                                            + wb * b_v[i, pl.ds(j, 16)])
                return carry

            lax.fori_loop(0, CB, body, 0)
            pltpu.sync_copy(c_v, out_hbm.at[pl.ds(base, CB)])

    return k(h2, ys, pos0, pos1, w0, w1)


def kernel(hidden_states, pre_ln_gamma, post_ln_gamma, Wqkv, Wo, Wg, W1, W2):
    Wqkv_b = Wqkv.astype(jnp.bfloat16)
    Wo_b = Wo.astype(jnp.bfloat16)
    W1_b = W1.astype(jnp.bfloat16)
    W2_b = W2.astype(jnp.bfloat16)

    qkv = _qkv(hidden_states, pre_ln_gamma.reshape(1, D), Wqkv_b)
    attn = _attn(qkv)
    h2, hn, sel, w, rank, counts = _oproj_router(attn, hidden_states, Wo_b,
                                                 post_ln_gamma.reshape(1, D), Wg)
    pos0, pos1, te, fill = _routing_plan(sel, rank, counts)
    xs = _dispatch_sc(hn, pos0, pos1)
    ys = _gmm(te, fill, xs, W1_b, W2_b)
    w0x = jnp.broadcast_to(w[:, 0:1], (S, 16))
    w1x = jnp.broadcast_to(w[:, 1:2], (S, 16))
    out = _combine_sc(h2, ys, pos0, pos1, w0x, w1x)
    return out


# speedup vs baseline: 1.7415x; 1.7415x over previous
import functools

import jax
import jax.numpy as jnp
from jax import lax
from jax.experimental import pallas as pl
from jax.experimental.pallas import tpu as pltpu
from jax.experimental.pallas import tpu_sc as plsc

S = 2048
D = 1024
H = 16
DH = 64
E = 8
K = 2
F = 512
EPS = 1e-6

TQ = 256
TM = 256
NPAD = 6144
NT = NPAD // TM

_INTERPRET = False


def _rms(x, g):
    v = jnp.mean(x * x, axis=-1, keepdims=True)
    return x * lax.rsqrt(v + EPS) * g


def _qkv_kernel(h_ref, g_ref, w_ref, qkv_ref):
    x = _rms(h_ref[...], g_ref[...])
    qkv_ref[...] = jnp.dot(x.astype(jnp.bfloat16), w_ref[...],
                           preferred_element_type=jnp.float32).astype(jnp.bfloat16)


def _qkv(hidden, gamma, Wqkv_b):
    return pl.pallas_call(
        _qkv_kernel,
        grid=(S // TQ,),
        in_specs=[
            pl.BlockSpec((TQ, D), lambda i: (i, 0)),
            pl.BlockSpec((1, D), lambda i: (0, 0)),
            pl.BlockSpec((D, 3 * D), lambda i: (0, 0)),
        ],
        out_specs=pl.BlockSpec((TQ, 3 * D), lambda i: (i, 0)),
        out_shape=jax.ShapeDtypeStruct((S, 3 * D), jnp.bfloat16),
        interpret=_INTERPRET,
    )(hidden, gamma, Wqkv_b)


def _attn_kernel(q_ref, kv_ref, o_ref):
    qb = pl.program_id(0)
    rows = qb * TQ + lax.broadcasted_iota(jnp.int32, (TQ, S), 0)
    cols = lax.broadcasted_iota(jnp.int32, (TQ, S), 1)
    causal = cols <= rows
    outs = []
    for h in range(H):
        q = q_ref[:, h * DH:(h + 1) * DH]
        k = kv_ref[:, D + h * DH:D + (h + 1) * DH]
        v = kv_ref[:, 2 * D + h * DH:2 * D + (h + 1) * DH]
        s = lax.dot_general(q, k, (((1,), (1,)), ((), ())),
                            preferred_element_type=jnp.float32)
        s = s * (1.0 / 8.0)
        s = jnp.where(causal, s, jnp.finfo(jnp.float32).min)
        m = jnp.max(s, axis=-1, keepdims=True)
        p = jnp.exp(s - m)
        l = jnp.sum(p, axis=-1, keepdims=True)
        outs.append(jnp.dot(p.astype(jnp.bfloat16), v,
                            preferred_element_type=jnp.float32) / l)
    o_ref[...] = jnp.concatenate(outs, axis=1).astype(jnp.bfloat16)


def _attn(qkv):
    return pl.pallas_call(
        _attn_kernel,
        grid=(S // TQ,),
        in_specs=[
            pl.BlockSpec((TQ, 3 * D), lambda qb: (qb, 0)),
            pl.BlockSpec((S, 3 * D), lambda qb: (0, 0)),
        ],
        out_specs=pl.BlockSpec((TQ, D), lambda qb: (qb, 0)),
        out_shape=jax.ShapeDtypeStruct((S, D), jnp.bfloat16),
        interpret=_INTERPRET,
    )(qkv, qkv)


def _oproj_kernel(a_ref, res_ref, wo_ref, g_ref, wg_ref,
                  h2_ref, hn_ref, sel_ref, w_ref, rank_ref, cnt_ref, run_ref):
    step = pl.program_id(0)

    @pl.when(step == 0)
    def _init():
        run_ref[...] = jnp.zeros_like(run_ref)

    o = jnp.dot(a_ref[...].astype(jnp.bfloat16), wo_ref[...],
                preferred_element_type=jnp.float32)
    h2 = res_ref[...] + o
    h2_ref[...] = h2
    hn = _rms(h2, g_ref[...])
    hn_ref[...] = hn
    logits = jnp.dot(hn.astype(jnp.bfloat16), wg_ref[...].astype(jnp.bfloat16),
                     preferred_element_type=jnp.float32)
    m = jnp.max(logits, axis=-1, keepdims=True)
    p = jnp.exp(logits - m)
    probs = p / jnp.sum(p, axis=-1, keepdims=True)
    idx = lax.broadcasted_iota(jnp.int32, (TQ, E), 1)
    m1 = jnp.max(probs, axis=-1, keepdims=True)
    i1 = jnp.min(jnp.where(probs == m1, idx, E), axis=-1, keepdims=True)
    probs2 = jnp.where(idx == i1, -1.0, probs)
    m2 = jnp.max(probs2, axis=-1, keepdims=True)
    i2 = jnp.min(jnp.where(probs2 == m2, idx, E), axis=-1, keepdims=True)
    tot = m1 + m2
    w1 = m1 / tot
    w2 = m2 / tot
    sel_ref[...] = jnp.where(idx == 0, i1, jnp.where(idx == 1, i2, 0))
    w_ref[...] = jnp.where(idx == 0, w1, jnp.where(idx == 1, w2, 0.0))

    oh1 = (idx == i1).astype(jnp.float32)
    oh2 = (idx == i2).astype(jnp.float32)
    oh = (oh1 + oh2).astype(jnp.bfloat16)
    r = lax.broadcasted_iota(jnp.int32, (TQ, TQ), 0)
    c = lax.broadcasted_iota(jnp.int32, (TQ, TQ), 1)
    tri = (c < r).astype(jnp.bfloat16)
    excl = jnp.dot(tri, oh, preferred_element_type=jnp.float32)
    run = run_ref[...]
    rank1 = jnp.sum((excl + run) * oh1, axis=-1, keepdims=True)
    rank2 = jnp.sum((excl + run) * oh2, axis=-1, keepdims=True)
    rank_ref[...] = jnp.where(idx == 0, rank1, jnp.where(idx == 1, rank2, 0.0))
    new_run = run + jnp.sum(oh1 + oh2, axis=0, keepdims=True)
    run_ref[...] = new_run
    cnt_ref[...] = new_run


def _oproj_router(attn, hidden, Wo_b, gamma, Wg):
    return pl.pallas_call(
        _oproj_kernel,
        grid=(S // TQ,),
        in_specs=[
            pl.BlockSpec((TQ, D), lambda i: (i, 0)),
            pl.BlockSpec((TQ, D), lambda i: (i, 0)),
            pl.BlockSpec((D, D), lambda i: (0, 0)),
            pl.BlockSpec((1, D), lambda i: (0, 0)),
            pl.BlockSpec((D, E), lambda i: (0, 0)),
        ],
        out_specs=[
            pl.BlockSpec((TQ, D), lambda i: (i, 0)),
            pl.BlockSpec((TQ, D), lambda i: (i, 0)),
            pl.BlockSpec((TQ, E), lambda i: (i, 0)),
            pl.BlockSpec((TQ, E), lambda i: (i, 0)),
            pl.BlockSpec((TQ, E), lambda i: (i, 0)),
            pl.BlockSpec((1, E), lambda i: (0, 0)),
        ],
        out_shape=[
            jax.ShapeDtypeStruct((S, D), jnp.float32),
            jax.ShapeDtypeStruct((S, D), jnp.float32),
            jax.ShapeDtypeStruct((S, E), jnp.int32),
            jax.ShapeDtypeStruct((S, E), jnp.float32),
            jax.ShapeDtypeStruct((S, E), jnp.float32),
            jax.ShapeDtypeStruct((1, E), jnp.float32),
        ],
        scratch_shapes=[pltpu.VMEM((1, E), jnp.float32)],
        interpret=_INTERPRET,
    )(attn, hidden, Wo_b, gamma, Wg)


def _gmm_kernel(te_ref, fill_ref, xs_ref, w1_ref, w2_ref, ys_ref):
    del te_ref
    t = pl.program_id(0)
    x = xs_ref[...].astype(jnp.bfloat16)
    gu = jnp.dot(x, w1_ref[...], preferred_element_type=jnp.float32)
    g = gu[:, :F]
    u = gu[:, F:]
    act = (g / (1.0 + jnp.exp(-g))) * u
    y = jnp.dot(act.astype(jnp.bfloat16), w2_ref[...],
                preferred_element_type=jnp.float32)
    valid = lax.broadcasted_iota(jnp.int32, (TM, 1), 0) < fill_ref[t]
    ys_ref[...] = jnp.where(valid, y, 0.0)


def _gmm(tile_expert, tile_fill, xs, W1_b, W2_b):
    grid_spec = pltpu.PrefetchScalarGridSpec(
        num_scalar_prefetch=2,
        grid=(NT,),
        in_specs=[
            pl.BlockSpec((TM, D), lambda t, te, fl: (t, 0)),
            pl.BlockSpec((D, 2 * F), lambda t, te, fl: (te[t], 0)),
            pl.BlockSpec((F, D), lambda t, te, fl: (te[t], 0)),
        ],
        out_specs=pl.BlockSpec((TM, D), lambda t, te, fl: (t, 0)),
    )
    return pl.pallas_call(
        _gmm_kernel,
        grid_spec=grid_spec,
        out_shape=jax.ShapeDtypeStruct((NPAD, D), jnp.float32),
        interpret=_INTERPRET,
    )(tile_expert, tile_fill, xs,
      W1_b.reshape(E * D, 2 * F), W2_b.reshape(E * F, D))


def _routing_plan(sel, rank, counts):
    counts_i = counts.reshape(E).astype(jnp.int32)
    gpad = ((counts_i + TM - 1) // TM) * TM
    startp = jnp.concatenate([jnp.zeros((1,), jnp.int32),
                              jnp.cumsum(gpad)[:-1].astype(jnp.int32)])
    pos0 = startp[sel[:, 0]] + rank[:, 0].astype(jnp.int32)
    pos1 = startp[sel[:, 1]] + rank[:, 1].astype(jnp.int32)
    tstart = jnp.arange(NT, dtype=jnp.int32) * TM
    te = jnp.sum(tstart[:, None] >= startp[None, :], axis=1) - 1
    te = jnp.clip(te, 0, E - 1).astype(jnp.int32)
    fill = jnp.clip(counts_i[te] - (tstart - startp[te]), 0, TM).astype(jnp.int32)
    return pos0, pos1, te, fill


NC = 2
NS = 16
NW = NC * NS
CHUNK = S // NW
CB = 32


def _dispatch_sc(hn, pos0, pos1):
    mesh = plsc.VectorSubcoreMesh(core_axis_name="c", subcore_axis_name="s")

    @functools.partial(
        pl.kernel, mesh=mesh,
        out_type=jax.ShapeDtypeStruct((NPAD, D), jnp.float32),
        scratch_types=[pltpu.VMEM((CHUNK,), jnp.int32),
                       pltpu.VMEM((CHUNK,), jnp.int32),
                       pltpu.VMEM((CHUNK, D), jnp.float32),
                       pltpu.SemaphoreType.DMA],
    )
    def k(hn_hbm, p0_hbm, p1_hbm, xs_hbm, i0_v, i1_v, rows_v, sem):
        wid = lax.axis_index("s") * NC + lax.axis_index("c")
        base = wid * CHUNK
        pltpu.sync_copy(p0_hbm.at[pl.ds(base, CHUNK)], i0_v)
        pltpu.sync_copy(p1_hbm.at[pl.ds(base, CHUNK)], i1_v)
        pltpu.sync_copy(hn_hbm.at[pl.ds(base, CHUNK)], rows_v)
        c0 = pltpu.async_copy(rows_v, xs_hbm.at[i0_v], sem)
        c1 = pltpu.async_copy(rows_v, xs_hbm.at[i1_v], sem)
        c0.wait()
        c1.wait()

    return k(hn, pos0, pos1)


def _combine_sc(h2, ys, pos0, pos1, w0, w1):
    mesh = plsc.VectorSubcoreMesh(core_axis_name="c", subcore_axis_name="s")

    @functools.partial(
        pl.kernel, mesh=mesh,
        out_type=jax.ShapeDtypeStruct((S, D), jnp.float32),
        scratch_types=[pltpu.VMEM((CB,), jnp.int32),
                       pltpu.VMEM((CB,), jnp.int32),
                       pltpu.VMEM((CB, 16), jnp.float32),
                       pltpu.VMEM((CB, 16), jnp.float32),
                       pltpu.VMEM((CB, D), jnp.float32),
                       pltpu.VMEM((CB, D), jnp.float32),
                       pltpu.VMEM((CB, D), jnp.float32),
                       pltpu.SemaphoreType.DMA],
    )
    def k(h2_hbm, ys_hbm, p0_hbm, p1_hbm, w0_hbm, w1_hbm, out_hbm,
          i0_v, i1_v, w0_v, w1_v, a_v, b_v, c_v, sem):
        wid = lax.axis_index("s") * NC + lax.axis_index("c")
        for sub in range(CHUNK // CB):
            base = wid * CHUNK + sub * CB
            pltpu.sync_copy(p0_hbm.at[pl.ds(base, CB)], i0_v)
            pltpu.sync_copy(p1_hbm.at[pl.ds(base, CB)], i1_v)
            pltpu.sync_copy(w0_hbm.at[pl.ds(base, CB)], w0_v)
            pltpu.sync_copy(w1_hbm.at[pl.ds(base, CB)], w1_v)
            g0 = pltpu.async_copy(ys_hbm.at[i0_v], a_v, sem)
            g1 = pltpu.async_copy(ys_hbm.at[i1_v], b_v, sem)
            pltpu.sync_copy(h2_hbm.at[pl.ds(base, CB)], c_v)
            g0.wait()
            g1.wait()

            def body(i, carry):
                wa = w0_v[i, :]
                wb = w1_v[i, :]
                for j in range(0, D, 16):
                    c_v[i, pl.ds(j, 16)] = (c_v[i, pl.ds(j, 16)]
                                            + wa * a_v[i, pl.ds(j, 16)]
                                            + wb * b_v[i, pl.ds(j, 16)])
                return carry

            lax.fori_loop(0, CB, body, 0)
            pltpu.sync_copy(c_v, out_hbm.at[pl.ds(base, CB)])

    return k(h2, ys, pos0, pos1, w0, w1)


def kernel(hidden_states, pre_ln_gamma, post_ln_gamma, Wqkv, Wo, Wg, W1, W2):
    Wqkv_b = Wqkv.astype(jnp.bfloat16)
    Wo_b = Wo.astype(jnp.bfloat16)
    W1_b = W1.astype(jnp.bfloat16)
    W2_b = W2.astype(jnp.bfloat16)

    qkv = _qkv(hidden_states, pre_ln_gamma.reshape(1, D), Wqkv_b)
    attn = _attn(qkv)
    h2, hn, sel, w, rank, counts = _oproj_router(attn, hidden_states, Wo_b,
                                                 post_ln_gamma.reshape(1, D), Wg)
    pos0, pos1, te, fill = _routing_plan(sel, rank, counts)
    xs = _dispatch_sc(hn, pos0, pos1)
    ys = _gmm(te, fill, xs, W1_b, W2_b)
    w0x = jnp.broadcast_to(w[:, 0:1], (S, 16))
    w1x = jnp.broadcast_to(w[:, 1:2], (S, 16))
    out = _combine_sc(h2, ys, pos0, pos1, w0x, w1x)
    return out


# speedup vs baseline: 1.8734x; 1.0757x over previous
import functools

import jax
import jax.numpy as jnp
from jax import lax
from jax.experimental import pallas as pl
from jax.experimental.pallas import tpu as pltpu
from jax.experimental.pallas import tpu_sc as plsc

S = 2048
D = 1024
H = 16
DH = 64
E = 8
K = 2
F = 512
EPS = 1e-6

TQ = 256
TM = 256
NPAD = 6144
NT = NPAD // TM

_INTERPRET = False


def _rms(x, g):
    v = jnp.mean(x * x, axis=-1, keepdims=True)
    return x * lax.rsqrt(v + EPS) * g


def _qkv_kernel(h_ref, g_ref, w_ref, qkv_ref):
    x = _rms(h_ref[...], g_ref[...])
    qkv_ref[...] = jnp.dot(x.astype(jnp.bfloat16), w_ref[...].astype(jnp.bfloat16),
                           preferred_element_type=jnp.float32).astype(jnp.bfloat16)


def _qkv(hidden, gamma, Wqkv_b):
    return pl.pallas_call(
        _qkv_kernel,
        grid=(S // TQ,),
        in_specs=[
            pl.BlockSpec((TQ, D), lambda i: (i, 0)),
            pl.BlockSpec((1, D), lambda i: (0, 0)),
            pl.BlockSpec((D, 3 * D), lambda i: (0, 0)),
        ],
        out_specs=pl.BlockSpec((TQ, 3 * D), lambda i: (i, 0)),
        out_shape=jax.ShapeDtypeStruct((S, 3 * D), jnp.bfloat16),
        interpret=_INTERPRET,
    )(hidden, gamma, Wqkv_b)


def _attn_kernel(q_ref, kv_ref, o_ref):
    qb = pl.program_id(0)
    rows = qb * TQ + lax.broadcasted_iota(jnp.int32, (TQ, S), 0)
    cols = lax.broadcasted_iota(jnp.int32, (TQ, S), 1)
    causal = cols <= rows
    outs = []
    for h in range(H):
        q = q_ref[:, h * DH:(h + 1) * DH]
        k = kv_ref[:, D + h * DH:D + (h + 1) * DH]
        v = kv_ref[:, 2 * D + h * DH:2 * D + (h + 1) * DH]
        s = lax.dot_general(q, k, (((1,), (1,)), ((), ())),
                            preferred_element_type=jnp.float32)
        s = s * (1.0 / 8.0)
        s = jnp.where(causal, s, jnp.finfo(jnp.float32).min)
        m = jnp.max(s, axis=-1, keepdims=True)
        p = jnp.exp(s - m)
        l = jnp.sum(p, axis=-1, keepdims=True)
        outs.append(jnp.dot(p.astype(jnp.bfloat16), v,
                            preferred_element_type=jnp.float32) / l)
    o_ref[...] = jnp.concatenate(outs, axis=1).astype(jnp.bfloat16)


def _attn(qkv):
    return pl.pallas_call(
        _attn_kernel,
        grid=(S // TQ,),
        in_specs=[
            pl.BlockSpec((TQ, 3 * D), lambda qb: (qb, 0)),
            pl.BlockSpec((S, 3 * D), lambda qb: (0, 0)),
        ],
        out_specs=pl.BlockSpec((TQ, D), lambda qb: (qb, 0)),
        out_shape=jax.ShapeDtypeStruct((S, D), jnp.bfloat16),
        interpret=_INTERPRET,
    )(qkv, qkv)


def _oproj_kernel(a_ref, res_ref, wo_ref, g_ref, wg_ref,
                  h2_ref, hn_ref, sel_ref, w_ref, rank_ref, cnt_ref, run_ref):
    step = pl.program_id(0)

    @pl.when(step == 0)
    def _init():
        run_ref[...] = jnp.zeros_like(run_ref)

    o = jnp.dot(a_ref[...].astype(jnp.bfloat16), wo_ref[...].astype(jnp.bfloat16),
                preferred_element_type=jnp.float32)
    h2 = res_ref[...] + o
    h2_ref[...] = h2
    hn = _rms(h2, g_ref[...])
    hn_ref[...] = hn
    logits = jnp.dot(hn.astype(jnp.bfloat16), wg_ref[...].astype(jnp.bfloat16),
                     preferred_element_type=jnp.float32)
    m = jnp.max(logits, axis=-1, keepdims=True)
    p = jnp.exp(logits - m)
    probs = p / jnp.sum(p, axis=-1, keepdims=True)
    idx = lax.broadcasted_iota(jnp.int32, (TQ, E), 1)
    m1 = jnp.max(probs, axis=-1, keepdims=True)
    i1 = jnp.min(jnp.where(probs == m1, idx, E), axis=-1, keepdims=True)
    probs2 = jnp.where(idx == i1, -1.0, probs)
    m2 = jnp.max(probs2, axis=-1, keepdims=True)
    i2 = jnp.min(jnp.where(probs2 == m2, idx, E), axis=-1, keepdims=True)
    tot = m1 + m2
    w1 = m1 / tot
    w2 = m2 / tot
    sel_ref[...] = jnp.where(idx == 0, i1, jnp.where(idx == 1, i2, 0))
    w_ref[...] = jnp.where(idx == 0, w1, jnp.where(idx == 1, w2, 0.0))

    oh1 = (idx == i1).astype(jnp.float32)
    oh2 = (idx == i2).astype(jnp.float32)
    oh = (oh1 + oh2).astype(jnp.bfloat16)
    r = lax.broadcasted_iota(jnp.int32, (TQ, TQ), 0)
    c = lax.broadcasted_iota(jnp.int32, (TQ, TQ), 1)
    tri = (c < r).astype(jnp.bfloat16)
    excl = jnp.dot(tri, oh, preferred_element_type=jnp.float32)
    run = run_ref[...]
    rank1 = jnp.sum((excl + run) * oh1, axis=-1, keepdims=True)
    rank2 = jnp.sum((excl + run) * oh2, axis=-1, keepdims=True)
    rank_ref[...] = jnp.where(idx == 0, rank1, jnp.where(idx == 1, rank2, 0.0))
    new_run = run + jnp.sum(oh1 + oh2, axis=0, keepdims=True)
    run_ref[...] = new_run
    cnt_ref[...] = new_run


def _oproj_router(attn, hidden, Wo_b, gamma, Wg):
    return pl.pallas_call(
        _oproj_kernel,
        grid=(S // TQ,),
        in_specs=[
            pl.BlockSpec((TQ, D), lambda i: (i, 0)),
            pl.BlockSpec((TQ, D), lambda i: (i, 0)),
            pl.BlockSpec((D, D), lambda i: (0, 0)),
            pl.BlockSpec((1, D), lambda i: (0, 0)),
            pl.BlockSpec((D, E), lambda i: (0, 0)),
        ],
        out_specs=[
            pl.BlockSpec((TQ, D), lambda i: (i, 0)),
            pl.BlockSpec((TQ, D), lambda i: (i, 0)),
            pl.BlockSpec((TQ, E), lambda i: (i, 0)),
            pl.BlockSpec((TQ, E), lambda i: (i, 0)),
            pl.BlockSpec((TQ, E), lambda i: (i, 0)),
            pl.BlockSpec((1, E), lambda i: (0, 0)),
        ],
        out_shape=[
            jax.ShapeDtypeStruct((S, D), jnp.float32),
            jax.ShapeDtypeStruct((S, D), jnp.float32),
            jax.ShapeDtypeStruct((S, E), jnp.int32),
            jax.ShapeDtypeStruct((S, E), jnp.float32),
            jax.ShapeDtypeStruct((S, E), jnp.float32),
            jax.ShapeDtypeStruct((1, E), jnp.float32),
        ],
        scratch_shapes=[pltpu.VMEM((1, E), jnp.float32)],
        interpret=_INTERPRET,
    )(attn, hidden, Wo_b, gamma, Wg)


def _gmm_kernel(te_ref, fill_ref, xs_ref, w1_ref, w2_ref, ys_ref):
    del te_ref
    t = pl.program_id(0)
    x = xs_ref[...].astype(jnp.bfloat16)
    gu = jnp.dot(x, w1_ref[...].astype(jnp.bfloat16),
                 preferred_element_type=jnp.float32)
    g = gu[:, :F]
    u = gu[:, F:]
    act = (g / (1.0 + jnp.exp(-g))) * u
    y = jnp.dot(act.astype(jnp.bfloat16), w2_ref[...].astype(jnp.bfloat16),
                preferred_element_type=jnp.float32)
    valid = lax.broadcasted_iota(jnp.int32, (TM, 1), 0) < fill_ref[t]
    ys_ref[...] = jnp.where(valid, y, 0.0)


def _gmm(tile_expert, tile_fill, xs, W1_b, W2_b):
    grid_spec = pltpu.PrefetchScalarGridSpec(
        num_scalar_prefetch=2,
        grid=(NT,),
        in_specs=[
            pl.BlockSpec((TM, D), lambda t, te, fl: (t, 0)),
            pl.BlockSpec((D, 2 * F), lambda t, te, fl: (te[t], 0)),
            pl.BlockSpec((F, D), lambda t, te, fl: (te[t], 0)),
        ],
        out_specs=pl.BlockSpec((TM, D), lambda t, te, fl: (t, 0)),
    )
    return pl.pallas_call(
        _gmm_kernel,
        grid_spec=grid_spec,
        out_shape=jax.ShapeDtypeStruct((NPAD, D), jnp.float32),
        interpret=_INTERPRET,
    )(tile_expert, tile_fill, xs,
      W1_b.reshape(E * D, 2 * F), W2_b.reshape(E * F, D))


def _routing_plan(sel, rank, counts):
    counts_i = counts.reshape(E).astype(jnp.int32)
    gpad = ((counts_i + TM - 1) // TM) * TM
    startp = jnp.concatenate([jnp.zeros((1,), jnp.int32),
                              jnp.cumsum(gpad)[:-1].astype(jnp.int32)])
    pos0 = startp[sel[:, 0]] + rank[:, 0].astype(jnp.int32)
    pos1 = startp[sel[:, 1]] + rank[:, 1].astype(jnp.int32)
    tstart = jnp.arange(NT, dtype=jnp.int32) * TM
    te = jnp.sum(tstart[:, None] >= startp[None, :], axis=1) - 1
    te = jnp.clip(te, 0, E - 1).astype(jnp.int32)
    fill = jnp.clip(counts_i[te] - (tstart - startp[te]), 0, TM).astype(jnp.int32)
    return pos0, pos1, te, fill


NC = 2
NS = 16
NW = NC * NS
CHUNK = S // NW
CB = 32


def _dispatch_sc(hn, pos0, pos1):
    mesh = plsc.VectorSubcoreMesh(core_axis_name="c", subcore_axis_name="s")

    @functools.partial(
        pl.kernel, mesh=mesh,
        out_type=jax.ShapeDtypeStruct((NPAD, D), jnp.float32),
        scratch_types=[pltpu.VMEM((CHUNK,), jnp.int32),
                       pltpu.VMEM((CHUNK,), jnp.int32),
                       pltpu.VMEM((CHUNK, D), jnp.float32),
                       pltpu.SemaphoreType.DMA],
    )
    def k(hn_hbm, p0_hbm, p1_hbm, xs_hbm, i0_v, i1_v, rows_v, sem):
        wid = lax.axis_index("s") * NC + lax.axis_index("c")
        base = wid * CHUNK
        pltpu.sync_copy(p0_hbm.at[pl.ds(base, CHUNK)], i0_v)
        pltpu.sync_copy(p1_hbm.at[pl.ds(base, CHUNK)], i1_v)
        pltpu.sync_copy(hn_hbm.at[pl.ds(base, CHUNK)], rows_v)
        c0 = pltpu.async_copy(rows_v, xs_hbm.at[i0_v], sem)
        c1 = pltpu.async_copy(rows_v, xs_hbm.at[i1_v], sem)
        c0.wait()
        c1.wait()

    return k(hn, pos0, pos1)


def _combine_sc(h2, ys, pos0, pos1, w0, w1):
    mesh = plsc.VectorSubcoreMesh(core_axis_name="c", subcore_axis_name="s")

    @functools.partial(
        pl.kernel, mesh=mesh,
        out_type=jax.ShapeDtypeStruct((S, D), jnp.float32),
        scratch_types=[pltpu.VMEM((CB,), jnp.int32),
                       pltpu.VMEM((CB,), jnp.int32),
                       pltpu.VMEM((CB, 16), jnp.float32),
                       pltpu.VMEM((CB, 16), jnp.float32),
                       pltpu.VMEM((CB, D), jnp.float32),
                       pltpu.VMEM((CB, D), jnp.float32),
                       pltpu.VMEM((CB, D), jnp.float32),
                       pltpu.SemaphoreType.DMA],
    )
    def k(h2_hbm, ys_hbm, p0_hbm, p1_hbm, w0_hbm, w1_hbm, out_hbm,
          i0_v, i1_v, w0_v, w1_v, a_v, b_v, c_v, sem):
        wid = lax.axis_index("s") * NC + lax.axis_index("c")
        for sub in range(CHUNK // CB):
            base = wid * CHUNK + sub * CB
            pltpu.sync_copy(p0_hbm.at[pl.ds(base, CB)], i0_v)
            pltpu.sync_copy(p1_hbm.at[pl.ds(base, CB)], i1_v)
            pltpu.sync_copy(w0_hbm.at[pl.ds(base, CB)], w0_v)
            pltpu.sync_copy(w1_hbm.at[pl.ds(base, CB)], w1_v)
            g0 = pltpu.async_copy(ys_hbm.at[i0_v], a_v, sem)
            g1 = pltpu.async_copy(ys_hbm.at[i1_v], b_v, sem)
            pltpu.sync_copy(h2_hbm.at[pl.ds(base, CB)], c_v)
            g0.wait()
            g1.wait()

            def body(i, carry):
                wa = w0_v[i, :]
                wb = w1_v[i, :]
                for j in range(0, D, 16):
                    c_v[i, pl.ds(j, 16)] = (c_v[i, pl.ds(j, 16)]
                                            + wa * a_v[i, pl.ds(j, 16)]
                                            + wb * b_v[i, pl.ds(j, 16)])
                return carry

            lax.fori_loop(0, CB, body, 0)
            pltpu.sync_copy(c_v, out_hbm.at[pl.ds(base, CB)])

    return k(h2, ys, pos0, pos1, w0, w1)


def kernel(hidden_states, pre_ln_gamma, post_ln_gamma, Wqkv, Wo, Wg, W1, W2):
    Wqkv_b = Wqkv
    Wo_b = Wo
    W1_b = W1
    W2_b = W2

    qkv = _qkv(hidden_states, pre_ln_gamma.reshape(1, D), Wqkv_b)
    attn = _attn(qkv)
    h2, hn, sel, w, rank, counts = _oproj_router(attn, hidden_states, Wo_b,
                                                 post_ln_gamma.reshape(1, D), Wg)
    pos0, pos1, te, fill = _routing_plan(sel, rank, counts)
    xs = _dispatch_sc(hn, pos0, pos1)
    ys = _gmm(te, fill, xs, W1_b, W2_b)
    w0x = jnp.broadcast_to(w[:, 0:1], (S, 16))
    w1x = jnp.broadcast_to(w[:, 1:2], (S, 16))
    out = _combine_sc(h2, ys, pos0, pos1, w0x, w1x)
    return out
